# baseline (device time: 180844 ns/iter reference)
import jax
import jax.numpy as jnp
from jax import lax
from jax.experimental import pallas as pl
from jax.experimental.pallas import tpu as pltpu

N_DEV = 16
N_HOPS = 2 * (N_DEV - 1)
NSUB = 4
N_STR = 2 * NSUB


def kernel(x):
    m, n = x.shape
    cm = m // N_DEV
    sm = cm // (2 * NSUB)

    def body(x_ref, out_ref, comm_ref, xbuf_ref, send_sem, recv_sem,
             credit_sem, x_sem, out_sem):
        my = lax.axis_index("i")

        def logical_to_ring(l):
            p = lax.rem(l, 4)
            z = l // 4
            return jnp.where(
                p == 0, z,
                jnp.where(p == 1, 7 - z, jnp.where(p == 2, 8 + z, 15 - z)),
            )

        def ring_to_logical(q):
            q = lax.rem(q + 2 * N_DEV, N_DEV)
            return jnp.where(
                q < 4, 4 * q,
                jnp.where(
                    q < 8, 1 + 4 * (7 - q),
                    jnp.where(q < 12, 2 + 4 * (q - 8), 3 + 4 * (15 - q)),
                ),
            )

        r = logical_to_ring(my)
        left = ring_to_logical(r - 1)
        right = ring_to_logical(r + 1)

        streams = []
        for si in range(NSUB):
            for di, (dst, up, sgn) in enumerate(
                ((right, left, -1), (left, right, +1))
            ):
                streams.append({
                    "dst": dst, "up": up, "sgn": sgn,
                    "off": di * (cm // 2) + si * sm,
                })

        def chunk_rows(k, off):
            c = lax.rem(r + k + 2 * N_DEV, N_DEV)
            return pl.ds(c * cm + off, sm)

        def x_fetch(st, h):
            dd = streams[st]
            cp = pltpu.make_async_copy(
                x_ref.at[chunk_rows(dd["sgn"] * h, dd["off"]), :],
                xbuf_ref.at[st, h % 2],
                x_sem.at[st],
            )
            cp.start()
            return cp

        def descriptor(st, p):
            return pltpu.make_async_remote_copy(
                src_ref=comm_ref.at[st, 0],
                dst_ref=comm_ref.at[st, 1 + p],
                send_sem=send_sem.at[st],
                recv_sem=recv_sem.at[st, p],
                device_id=(streams[st]["dst"],),
                device_id_type=pl.DeviceIdType.MESH,
            )

        xcp = [x_fetch(st, 0) for st in range(N_STR)]

        barrier = pltpu.get_barrier_semaphore()
        for nbr in (left, right):
            pl.semaphore_signal(
                barrier, inc=1, device_id=(nbr,),
                device_id_type=pl.DeviceIdType.MESH,
            )
        pl.semaphore_wait(barrier, 2)

        prev = [None] * N_STR
        outcp = [None] * N_STR
        for h in range(N_HOPS):
            p = h % 3
            pp = (h - 1) % 3
            for st, dd in enumerate(streams):
                sgn, off = dd["sgn"], dd["off"]
                if h >= 1:
                    descriptor(st, pp).wait_recv()
                    prev[st].wait_send()
                if outcp[st] is not None:
                    outcp[st].wait()
                    outcp[st] = None
                if h <= N_DEV - 1:
                    pltpu.make_async_copy(
                        x_ref.at[chunk_rows(sgn * h, off), :],
                        xbuf_ref.at[st, h % 2], x_sem.at[st],
                    ).wait()
                if h == 0:
                    comm_ref[st, 0, :, :] = xbuf_ref[st, 0, :, :]
                elif h <= N_DEV - 1:
                    comm_ref[st, 0, :, :] = (
                        comm_ref[st, 1 + pp, :, :]
                        + xbuf_ref[st, h % 2, :, :]
                    )
                else:
                    comm_ref[st, 0, :, :] = comm_ref[st, 1 + pp, :, :]
                if h + 1 <= N_DEV - 1:
                    xcp[st] = x_fetch(st, h + 1)
                if h >= N_DEV - 1:
                    k = -sgn if h == N_DEV - 1 else sgn * (h - N_DEV)
                    cp = pltpu.make_async_copy(
                        comm_ref.at[st, 0],
                        out_ref.at[chunk_rows(k, off), :],
                        out_sem.at[st],
                    )
                    cp.start()
                    outcp[st] = cp
                if 2 <= h <= N_HOPS - 2:
                    pl.semaphore_signal(
                        credit_sem.at[st], inc=1, device_id=(dd["up"],),
                        device_id_type=pl.DeviceIdType.MESH,
                    )
                if h >= 3:
                    pl.semaphore_wait(credit_sem.at[st], 1)
                rdma = descriptor(st, p)
                rdma.start()
                prev[st] = rdma

        p = (N_HOPS - 1) % 3
        for st, dd in enumerate(streams):
            descriptor(st, p).wait_recv()
            outcp[st].wait()
            final = pltpu.make_async_copy(
                comm_ref.at[st, 1 + p],
                out_ref.at[chunk_rows(dd["sgn"] * (N_DEV - 2), dd["off"]), :],
                out_sem.at[st],
            )
            final.start()
            final.wait()
            prev[st].wait_send()

    return pl.pallas_call(
        body,
        out_shape=jax.ShapeDtypeStruct((m, n), jnp.float32),
        in_specs=[pl.BlockSpec(memory_space=pl.ANY)],
        out_specs=pl.BlockSpec(memory_space=pl.ANY),
        scratch_shapes=[
            pltpu.VMEM((N_STR, 4, sm, n), jnp.float32),
            pltpu.VMEM((N_STR, 2, sm, n), jnp.float32),
            pltpu.SemaphoreType.DMA((N_STR,)),
            pltpu.SemaphoreType.DMA((N_STR, 3)),
            pltpu.SemaphoreType.REGULAR((N_STR,)),
            pltpu.SemaphoreType.DMA((N_STR,)),
            pltpu.SemaphoreType.DMA((N_STR,)),
        ],
        compiler_params=pltpu.CompilerParams(collective_id=0),
    )(x)


# device time: 179761 ns/iter; 1.0060x vs baseline; 1.0060x over previous
import jax
import jax.numpy as jnp
from jax import lax
from jax.experimental import pallas as pl
from jax.experimental.pallas import tpu as pltpu

N_DEV = 16
N_HOPS = 2 * (N_DEV - 1)
NSUB = 2
N_STR = 2 * NSUB


def kernel(x):
    m, n = x.shape
    cm = m // N_DEV
    sm = cm // (2 * NSUB)

    def body(x_ref, out_ref, comm_ref, xbuf_ref, send_sem, recv_sem,
             credit_sem, x_sem, out_sem):
        my = lax.axis_index("i")

        def logical_to_ring(l):
            p = lax.rem(l, 4)
            z = l // 4
            return jnp.where(
                p == 0, z,
                jnp.where(p == 1, 7 - z, jnp.where(p == 2, 8 + z, 15 - z)),
            )

        def ring_to_logical(q):
            q = lax.rem(q + 2 * N_DEV, N_DEV)
            return jnp.where(
                q < 4, 4 * q,
                jnp.where(
                    q < 8, 1 + 4 * (7 - q),
                    jnp.where(q < 12, 2 + 4 * (q - 8), 3 + 4 * (15 - q)),
                ),
            )

        r = logical_to_ring(my)
        left = ring_to_logical(r - 1)
        right = ring_to_logical(r + 1)

        streams = []
        for si in range(NSUB):
            for di, (dst, up, sgn) in enumerate(
                ((right, left, -1), (left, right, +1))
            ):
                streams.append({
                    "dst": dst, "up": up, "sgn": sgn,
                    "off": di * (cm // 2) + si * sm,
                })

        def chunk_rows(k, off):
            c = lax.rem(r + k + 2 * N_DEV, N_DEV)
            return pl.ds(c * cm + off, sm)

        def x_fetch(st, h):
            dd = streams[st]
            cp = pltpu.make_async_copy(
                x_ref.at[chunk_rows(dd["sgn"] * h, dd["off"]), :],
                xbuf_ref.at[st, h % 2],
                x_sem.at[st],
            )
            cp.start()
            return cp

        def descriptor(st, p):
            return pltpu.make_async_remote_copy(
                src_ref=comm_ref.at[st, 0],
                dst_ref=comm_ref.at[st, 1 + p],
                send_sem=send_sem.at[st],
                recv_sem=recv_sem.at[st, p],
                device_id=(streams[st]["dst"],),
                device_id_type=pl.DeviceIdType.MESH,
            )

        xcp = [x_fetch(st, 0) for st in range(N_STR)]

        barrier = pltpu.get_barrier_semaphore()
        for nbr in (left, right):
            pl.semaphore_signal(
                barrier, inc=1, device_id=(nbr,),
                device_id_type=pl.DeviceIdType.MESH,
            )
        pl.semaphore_wait(barrier, 2)

        prev = [None] * N_STR
        outcp = [None] * N_STR
        for h in range(N_HOPS):
            p = h % 3
            pp = (h - 1) % 3
            for st, dd in enumerate(streams):
                sgn, off = dd["sgn"], dd["off"]
                if h >= 1:
                    descriptor(st, pp).wait_recv()
                    prev[st].wait_send()
                if outcp[st] is not None:
                    outcp[st].wait()
                    outcp[st] = None
                if h <= N_DEV - 1:
                    pltpu.make_async_copy(
                        x_ref.at[chunk_rows(sgn * h, off), :],
                        xbuf_ref.at[st, h % 2], x_sem.at[st],
                    ).wait()
                if h == 0:
                    comm_ref[st, 0, :, :] = xbuf_ref[st, 0, :, :]
                elif h <= N_DEV - 1:
                    comm_ref[st, 0, :, :] = (
                        comm_ref[st, 1 + pp, :, :]
                        + xbuf_ref[st, h % 2, :, :]
                    )
                else:
                    comm_ref[st, 0, :, :] = comm_ref[st, 1 + pp, :, :]
                if h + 1 <= N_DEV - 1:
                    xcp[st] = x_fetch(st, h + 1)
                if h >= N_DEV - 1:
                    k = -sgn if h == N_DEV - 1 else sgn * (h - N_DEV)
                    cp = pltpu.make_async_copy(
                        comm_ref.at[st, 0],
                        out_ref.at[chunk_rows(k, off), :],
                        out_sem.at[st],
                    )
                    cp.start()
                    outcp[st] = cp
                if 2 <= h <= N_HOPS - 2:
                    pl.semaphore_signal(
                        credit_sem.at[st], inc=1, device_id=(dd["up"],),
                        device_id_type=pl.DeviceIdType.MESH,
                    )
                if h >= 3:
                    pl.semaphore_wait(credit_sem.at[st], 1)
                rdma = descriptor(st, p)
                rdma.start()
                prev[st] = rdma

        p = (N_HOPS - 1) % 3
        for st, dd in enumerate(streams):
            descriptor(st, p).wait_recv()
            outcp[st].wait()
            final = pltpu.make_async_copy(
                comm_ref.at[st, 1 + p],
                out_ref.at[chunk_rows(dd["sgn"] * (N_DEV - 2), dd["off"]), :],
                out_sem.at[st],
            )
            final.start()
            final.wait()
            prev[st].wait_send()

    return pl.pallas_call(
        body,
        out_shape=jax.ShapeDtypeStruct((m, n), jnp.float32),
        in_specs=[pl.BlockSpec(memory_space=pl.ANY)],
        out_specs=pl.BlockSpec(memory_space=pl.ANY),
        scratch_shapes=[
            pltpu.VMEM((N_STR, 4, sm, n), jnp.float32),
            pltpu.VMEM((N_STR, 2, sm, n), jnp.float32),
            pltpu.SemaphoreType.DMA((N_STR,)),
            pltpu.SemaphoreType.DMA((N_STR, 3)),
            pltpu.SemaphoreType.REGULAR((N_STR,)),
            pltpu.SemaphoreType.DMA((N_STR,)),
            pltpu.SemaphoreType.DMA((N_STR,)),
        ],
        compiler_params=pltpu.CompilerParams(collective_id=0),
    )(x)
